# Initial kernel scaffold; baseline (speedup 1.0000x reference)
import jax
import jax.numpy as jnp
from jax.experimental import pallas as pl


def kernel(input, emb_weight):
    raise NotImplementedError("write your pallas kernel here")


# speedup vs baseline: 1.0734x; 1.0734x over previous
import jax
import jax.numpy as jnp
from jax import lax
from jax.experimental import pallas as pl
from jax.experimental.pallas import tpu as pltpu
from jax.experimental.pallas import tpu_sc as plsc

DIM = 32
NC = 2
NS = 16
NW = NC * NS
N = 16384 * 50
PER_W = N // NW
CHUNK = 512
N_CHUNKS = PER_W // CHUNK


def _gather_body(idx_hbm, table_hbm, out_hbm, idx_v, rows_v, sem):
    wid = lax.axis_index("s") * NC + lax.axis_index("c")
    base = wid * PER_W

    def chunk_body(g, carry):
        off = base + g * CHUNK
        pltpu.sync_copy(idx_hbm.at[pl.ds(off, CHUNK)], idx_v)
        pltpu.async_copy(table_hbm.at[idx_v], rows_v, sem).wait()
        pltpu.sync_copy(rows_v, out_hbm.at[pl.ds(off, CHUNK)])
        return carry

    lax.fori_loop(0, N_CHUNKS, chunk_body, 0)


@jax.jit
def kernel(input, emb_weight):
    idx = input.reshape(-1).astype(jnp.int32)
    mesh = plsc.VectorSubcoreMesh(core_axis_name="c", subcore_axis_name="s")
    out = pl.kernel(
        _gather_body,
        mesh=mesh,
        compiler_params=pltpu.CompilerParams(use_tc_tiling_on_sc=False),
        out_type=jax.ShapeDtypeStruct((N, DIM), jnp.float32),
        scratch_types=[
            pltpu.VMEM((CHUNK,), jnp.int32),
            pltpu.VMEM((CHUNK, DIM), jnp.float32),
            pltpu.SemaphoreType.DMA,
        ],
    )(idx, emb_weight)
    return out.reshape(input.shape[0], input.shape[1], DIM)


# speedup vs baseline: 1.1126x; 1.0365x over previous
import jax
import jax.numpy as jnp
from jax import lax
from jax.experimental import pallas as pl
from jax.experimental.pallas import tpu as pltpu
from jax.experimental.pallas import tpu_sc as plsc

DIM = 32
NC = 2
NS = 16
NW = NC * NS
N = 16384 * 50
PER_W = N // NW
CHUNK = 640
N_CHUNKS = PER_W // CHUNK
NBUF = 4
MAJORS = N_CHUNKS // NBUF


def _gather_body(idx_hbm, table_hbm, out_hbm, idx_v, *bufs_and_sems):
    rows = bufs_and_sems[:NBUF]
    gsem = bufs_and_sems[NBUF:2 * NBUF]
    osem = bufs_and_sems[2 * NBUF:3 * NBUF]

    wid = lax.axis_index("s") * NC + lax.axis_index("c")
    base = wid * PER_W
    pltpu.sync_copy(idx_hbm.at[pl.ds(base, PER_W)], idx_v)

    def idx_slice(g):
        return idx_v.at[pl.ds(g * CHUNK, CHUNK)]

    for b in range(NBUF):
        pltpu.async_copy(table_hbm.at[idx_slice(b)], rows[b], gsem[b])

    def major_body(m, carry):
        for b in range(NBUF):
            g = m * NBUF + b
            off = base + g * CHUNK
            pltpu.make_async_copy(
                table_hbm.at[idx_slice(g)], rows[b], gsem[b]).wait()
            pltpu.async_copy(rows[b], out_hbm.at[pl.ds(off, CHUNK)], osem[b])
            pltpu.make_async_copy(
                rows[b], out_hbm.at[pl.ds(off, CHUNK)], osem[b]).wait()
            pltpu.async_copy(table_hbm.at[idx_slice(g + NBUF)], rows[b],
                             gsem[b])
        return carry

    lax.fori_loop(0, MAJORS - 1, major_body, 0)

    for b in range(NBUF):
        g = (MAJORS - 1) * NBUF + b
        off = base + g * CHUNK
        pltpu.make_async_copy(
            table_hbm.at[idx_slice(g)], rows[b], gsem[b]).wait()
        pltpu.async_copy(rows[b], out_hbm.at[pl.ds(off, CHUNK)], osem[b])
    for b in range(NBUF):
        g = (MAJORS - 1) * NBUF + b
        off = base + g * CHUNK
        pltpu.make_async_copy(
            rows[b], out_hbm.at[pl.ds(off, CHUNK)], osem[b]).wait()


@jax.jit
def kernel(input, emb_weight):
    idx = input.reshape(-1).astype(jnp.int32)
    mesh = plsc.VectorSubcoreMesh(core_axis_name="c", subcore_axis_name="s")
    out = pl.kernel(
        _gather_body,
        mesh=mesh,
        compiler_params=pltpu.CompilerParams(use_tc_tiling_on_sc=False),
        out_type=jax.ShapeDtypeStruct((N, DIM), jnp.float32),
        scratch_types=[
            pltpu.VMEM((PER_W,), jnp.int32),
            *[pltpu.VMEM((CHUNK, DIM), jnp.float32) for _ in range(NBUF)],
            *[pltpu.SemaphoreType.DMA for _ in range(2 * NBUF)],
        ],
    )(idx, emb_weight)
    return out.reshape(input.shape[0], input.shape[1], DIM)


# speedup vs baseline: 1.6826x; 1.5123x over previous
import jax
import jax.numpy as jnp
from jax import lax
from jax.experimental import pallas as pl
from jax.experimental.pallas import tpu as pltpu
from jax.experimental.pallas import tpu_sc as plsc

VOCAB = 1000000
DIM = 32
B = 16384
L = 50
LPAD = 56
HALF = 16
NS = 16

WA = 128
NFULL = VOCAB // WA
ATAIL = VOCAB - NFULL * WA
APT = NFULL // NS
AXTRA = NFULL - APT * NS

CB = 128
CPL = B // CB
NLB = LPAD // 8
NSC = NLB * CPL
SPT = NSC // NS


def _body(idx_hbm, tbl_hbm, tail_hbm, out_hbm, scr,
          ain0, ain1, aout0, aout1, idxb0, idxb1,
          if0, if1, if2, if3, if4, if5, if6, if7,
          tm0, tm1, tm2, tm3, tm4, tm5, tm6, tm7,
          rows0, rows1, outt0, outt1, tin, tout,
          sai0, sai1, sao0, sao1, sid0, sid1, sg0, sg1, so0, so1):
    ain = (ain0, ain1)
    aout = (aout0, aout1)
    idxb = (idxb0, idxb1)
    idxf = (if0, if1, if2, if3, if4, if5, if6, if7)
    tokm = (tm0, tm1, tm2, tm3, tm4, tm5, tm6, tm7)
    rows = (rows0, rows1)
    outt = (outt0, outt1)
    sai = (sai0, sai1)
    sao = (sao0, sao1)
    sid = (sid0, sid1)
    sg = (sg0, sg1)
    so = (so0, so1)

    c = lax.axis_index("c")
    t = lax.axis_index("s")
    f0 = pl.multiple_of(c * HALF, 8)
    coff = c * VOCAB
    iota = lax.iota(jnp.int32, 16)
    fvecs = [jnp.full((16,), f, jnp.int32) for f in range(HALF)]

    def a_src(b):
        return tbl_hbm.at[pl.ds(f0, HALF),
                          pl.ds(pl.multiple_of(b * WA, 128), WA)]

    def a_issue(b, p):
        pltpu.async_copy(a_src(b), ain[p], sai[p])

    def a_wait_in(b, p):
        pltpu.make_async_copy(a_src(b), ain[p], sai[p]).wait()

    def a_dst(b):
        return scr.at[pl.ds(c * 125000 + b * (WA // 8), WA // 8)]

    def a_compute(p):
        def grp(g, carry):
            tok = g * 16 + iota
            for f in range(HALF):
                v = plsc.load_gather(ain[p], [fvecs[f], tok])
                plsc.store_scatter(
                    aout[p], [tok // 8, (tok % 8) * 16 + fvecs[f]], v)
            return carry
        lax.fori_loop(0, WA // 16, grp, 0)

    abase = t * APT
    a_issue(abase, 0)

    def a_major(m, carry):
        for p in range(2):
            j = m * 2 + p
            b = abase + j
            a_wait_in(b, p)

            @pl.when(j + 1 < APT)
            def _():
                a_issue(b + 1, 1 - p)

            a_compute(p)

            @pl.when(j >= 2)
            def _():
                pltpu.make_async_copy(aout[p], a_dst(b - 2), sao[p]).wait()

            pltpu.async_copy(aout[p], a_dst(b), sao[p])
        return carry

    lax.fori_loop(0, APT // 2, a_major, 0)
    pltpu.make_async_copy(aout[0], a_dst(abase + APT - 2), sao[0]).wait()
    pltpu.make_async_copy(aout[1], a_dst(abase + APT - 1), sao[1]).wait()

    @pl.when(t < AXTRA)
    def _():
        b = NFULL - AXTRA + t
        a_issue(b, 0)
        a_wait_in(b, 0)
        a_compute(0)
        pltpu.async_copy(aout[0], a_dst(b), sao[0]).wait()

    @pl.when(t == AXTRA)
    def _():
        pltpu.async_copy(tail_hbm, tin, sai[0]).wait()

        def grp(g, carry):
            tok = g * 16 + iota
            for f in range(HALF):
                v = plsc.load_gather(tin, [fvecs[f] + f0, tok])
                plsc.store_scatter(
                    tout, [tok // 8, (tok % 8) * 16 + fvecs[f]], v)
            return carry
        lax.fori_loop(0, ATAIL // 16, grp, 0)
        pltpu.async_copy(
            tout, scr.at[pl.ds(c * 125000 + 124992, ATAIL // 8)],
            sao[0]).wait()

    plsc.subcore_barrier()

    kbase = t * SPT

    def idx_src(k):
        lb = k // CPL
        bt = k - lb * CPL
        return idx_hbm.at[pl.ds(pl.multiple_of(lb * 8, 8), 8),
                          pl.ds(pl.multiple_of(bt * CB, 128), CB)]

    def out_dst(k, ll):
        lb = k // CPL
        bt = k - lb * CPL
        return out_hbm.at[lb * 8 + ll, pl.ds(f0, HALF),
                          pl.ds(pl.multiple_of(bt * CB, 128), CB)]

    def out_issue(q, k, ll):
        pltpu.async_copy(outt[q], out_dst(k, ll), so[q])

    def out_wait(q, k, ll):
        pltpu.make_async_copy(outt[q], out_dst(k, ll), so[q]).wait()

    def b_compute(q, ll):
        def grp(g, carry):
            s = g * 16 + iota
            mbase = plsc.load_gather(tokm[ll], [s]) * 16
            for f in range(HALF):
                v = plsc.load_gather(rows[q], [s, mbase + fvecs[f]])
                plsc.store_scatter(outt[q], [fvecs[f], s], v)
            return carry
        lax.fori_loop(0, CB // 16, grp, 0)

    def flatten(p):
        for ll in range(8):
            lvec = jnp.full((16,), ll, jnp.int32)

            def grp(g, carry):
                tok = g * 16 + iota
                v = plsc.load_gather(idxb[p], [lvec, tok])
                plsc.store_scatter(idxf[ll], [tok], c * 125000 + v // 8)
                plsc.store_scatter(tokm[ll], [tok], v % 8)
                return carry
            lax.fori_loop(0, CB // 16, grp, 0)

    pltpu.async_copy(idx_src(kbase), idxb[0], sid[0])

    def b_major(m, carry):
        for p in range(2):
            j = m * 2 + p
            k = kbase + j
            lb = k // CPL
            nvalid = jnp.where(lb * 8 + 8 <= L, 8, L - lb * 8)
            pltpu.make_async_copy(idx_src(k), idxb[p], sid[p]).wait()

            @pl.when(j + 1 < SPT)
            def _():
                pltpu.async_copy(idx_src(k + 1), idxb[1 - p], sid[1 - p])

            flatten(p)

            pltpu.async_copy(scr.at[idxf[0]], rows[0], sg[0])
            for ll in range(8):
                q = ll % 2

                if ll < 7:
                    @pl.when(ll + 1 < nvalid)
                    def _(ll=ll, q=q):
                        pltpu.async_copy(scr.at[idxf[ll + 1]], rows[1 - q],
                                         sg[1 - q])

                @pl.when(ll < nvalid)
                def _(ll=ll, q=q):
                    pltpu.make_async_copy(
                        scr.at[idxf[ll]], rows[q], sg[q]).wait()
                    b_compute(q, ll)

                    @pl.when((j > 0) | (ll >= 2))
                    def _(ll=ll, q=q):
                        out_wait(q, k, ll)

                    out_issue(q, k, ll)
        return carry

    lax.fori_loop(0, SPT // 2, b_major, 0)
    k_last = kbase + SPT - 1
    out_wait(0, k_last, 0)
    out_wait(1, k_last, 1)


@jax.jit
def kernel(input, emb_weight):
    idx_t = jnp.pad(input.T, ((0, LPAD - L), (0, 0)))
    tbl_t = emb_weight.T
    tail_t = jnp.pad(emb_weight.T[:, NFULL * WA:],
                     ((0, 0), (0, WA - ATAIL)))
    mesh = plsc.VectorSubcoreMesh(core_axis_name="c", subcore_axis_name="s")
    out = pl.kernel(
        _body,
        mesh=mesh,
        compiler_params=pltpu.CompilerParams(
            use_tc_tiling_on_sc=True, needs_layout_passes=False),
        out_type=jax.ShapeDtypeStruct((L, DIM, B), jnp.float32),
        scratch_types=[
            pltpu.HBM((250000, 128), jnp.float32),
            pltpu.VMEM((HALF, WA), jnp.float32),
            pltpu.VMEM((HALF, WA), jnp.float32),
            pltpu.VMEM((HALF, WA), jnp.float32),
            pltpu.VMEM((HALF, WA), jnp.float32),
            pltpu.VMEM((8, CB), jnp.int32),
            pltpu.VMEM((8, CB), jnp.int32),
            *[pltpu.VMEM((CB,), jnp.int32) for _ in range(16)],
            pltpu.VMEM((CB, 128), jnp.float32),
            pltpu.VMEM((CB, 128), jnp.float32),
            pltpu.VMEM((HALF, CB), jnp.float32),
            pltpu.VMEM((HALF, CB), jnp.float32),
            pltpu.VMEM((DIM, WA), jnp.float32),
            pltpu.VMEM((8, WA), jnp.float32),
            *[pltpu.SemaphoreType.DMA for _ in range(10)],
        ],
    )(idx_t, tbl_t, tail_t)
    return out.transpose(2, 0, 1)


# speedup vs baseline: 1.7946x; 1.0666x over previous
import jax
import jax.numpy as jnp
from jax import lax
from jax.experimental import pallas as pl
from jax.experimental.pallas import tpu as pltpu
from jax.experimental.pallas import tpu_sc as plsc

VOCAB = 1000000
DIM = 32
B = 16384
L = 50
LPAD = 56
HALF = 16
NS = 16

WA = 128
NFULL = VOCAB // WA
ATAIL = VOCAB - NFULL * WA
APT = NFULL // NS
AXTRA = NFULL - APT * NS

CB = 256
CPL = B // CB
NLB = LPAD // 8
NSC = NLB * CPL
SPT = NSC // NS


def _body(idx_hbm, tbl_hbm, tail_hbm, out_hbm, scr,
          ain0, ain1, aout0, aout1, idxb0, idxb1,
          if0, if1, if2, if3, if4, if5, if6, if7,
          tm0, tm1, tm2, tm3, tm4, tm5, tm6, tm7,
          rows0, rows1, outt0, outt1, tin, tout,
          sai0, sai1, sao0, sao1, sid0, sid1, sg0, sg1, so0, so1):
    ain = (ain0, ain1)
    aout = (aout0, aout1)
    idxb = (idxb0, idxb1)
    idxf = (if0, if1, if2, if3, if4, if5, if6, if7)
    tokm = (tm0, tm1, tm2, tm3, tm4, tm5, tm6, tm7)
    rows = (rows0, rows1)
    outt = (outt0, outt1)
    sai = (sai0, sai1)
    sao = (sao0, sao1)
    sid = (sid0, sid1)
    sg = (sg0, sg1)
    so = (so0, so1)

    c = lax.axis_index("c")
    t = lax.axis_index("s")
    f0 = pl.multiple_of(c * HALF, 8)
    coff = c * VOCAB
    iota = lax.iota(jnp.int32, 16)
    fvecs = [jnp.full((16,), f, jnp.int32) for f in range(HALF)]

    def a_src(b):
        return tbl_hbm.at[pl.ds(f0, HALF),
                          pl.ds(pl.multiple_of(b * WA, 128), WA)]

    def a_issue(b, p):
        pltpu.async_copy(a_src(b), ain[p], sai[p])

    def a_wait_in(b, p):
        pltpu.make_async_copy(a_src(b), ain[p], sai[p]).wait()

    def a_dst(b):
        return scr.at[pl.ds(c * 125000 + b * (WA // 8), WA // 8)]

    def a_compute(p):
        def grp(g, carry):
            tok = g * 16 + iota
            for f in range(HALF):
                v = plsc.load_gather(ain[p], [fvecs[f], tok])
                plsc.store_scatter(
                    aout[p], [tok // 8, (tok % 8) * 16 + fvecs[f]], v)
            return carry
        lax.fori_loop(0, WA // 16, grp, 0)

    abase = t * APT
    a_issue(abase, 0)

    def a_major(m, carry):
        for p in range(2):
            j = m * 2 + p
            b = abase + j
            a_wait_in(b, p)

            @pl.when(j + 1 < APT)
            def _():
                a_issue(b + 1, 1 - p)

            a_compute(p)

            @pl.when(j >= 2)
            def _():
                pltpu.make_async_copy(aout[p], a_dst(b - 2), sao[p]).wait()

            pltpu.async_copy(aout[p], a_dst(b), sao[p])
        return carry

    lax.fori_loop(0, APT // 2, a_major, 0)
    pltpu.make_async_copy(aout[0], a_dst(abase + APT - 2), sao[0]).wait()
    pltpu.make_async_copy(aout[1], a_dst(abase + APT - 1), sao[1]).wait()

    @pl.when(t < AXTRA)
    def _():
        b = NFULL - AXTRA + t
        a_issue(b, 0)
        a_wait_in(b, 0)
        a_compute(0)
        pltpu.async_copy(aout[0], a_dst(b), sao[0]).wait()

    @pl.when(t == AXTRA)
    def _():
        pltpu.async_copy(tail_hbm, tin, sai[0]).wait()

        def grp(g, carry):
            tok = g * 16 + iota
            for f in range(HALF):
                v = plsc.load_gather(tin, [fvecs[f] + f0, tok])
                plsc.store_scatter(
                    tout, [tok // 8, (tok % 8) * 16 + fvecs[f]], v)
            return carry
        lax.fori_loop(0, ATAIL // 16, grp, 0)
        pltpu.async_copy(
            tout, scr.at[pl.ds(c * 125000 + 124992, ATAIL // 8)],
            sao[0]).wait()

    plsc.subcore_barrier()

    kbase = t * SPT

    def idx_src(k):
        lb = k // CPL
        bt = k - lb * CPL
        return idx_hbm.at[pl.ds(pl.multiple_of(lb * 8, 8), 8),
                          pl.ds(pl.multiple_of(bt * CB, 128), CB)]

    def out_dst(k, ll):
        lb = k // CPL
        bt = k - lb * CPL
        return out_hbm.at[lb * 8 + ll, pl.ds(f0, HALF),
                          pl.ds(pl.multiple_of(bt * CB, 128), CB)]

    def out_issue(q, k, ll):
        pltpu.async_copy(outt[q], out_dst(k, ll), so[q])

    def out_wait(q, k, ll):
        pltpu.make_async_copy(outt[q], out_dst(k, ll), so[q]).wait()

    def b_compute(q, ll):
        def grp(g, carry):
            s = g * 16 + iota
            mbase = plsc.load_gather(tokm[ll], [s]) * 16
            for f in range(HALF):
                v = plsc.load_gather(rows[q], [s, mbase + fvecs[f]])
                plsc.store_scatter(outt[q], [fvecs[f], s], v)
            return carry
        lax.fori_loop(0, CB // 16, grp, 0)

    def flatten(p):
        for ll in range(8):
            lvec = jnp.full((16,), ll, jnp.int32)

            def grp(g, carry):
                tok = g * 16 + iota
                v = plsc.load_gather(idxb[p], [lvec, tok])
                plsc.store_scatter(idxf[ll], [tok], c * 125000 + v // 8)
                plsc.store_scatter(tokm[ll], [tok], v % 8)
                return carry
            lax.fori_loop(0, CB // 16, grp, 0)

    pltpu.async_copy(idx_src(kbase), idxb[0], sid[0])

    def b_major(m, carry):
        for p in range(2):
            j = m * 2 + p
            k = kbase + j
            lb = k // CPL
            nvalid = jnp.where(lb * 8 + 8 <= L, 8, L - lb * 8)
            pltpu.make_async_copy(idx_src(k), idxb[p], sid[p]).wait()

            @pl.when(j + 1 < SPT)
            def _():
                pltpu.async_copy(idx_src(k + 1), idxb[1 - p], sid[1 - p])

            flatten(p)

            pltpu.async_copy(scr.at[idxf[0]], rows[0], sg[0])
            for ll in range(8):
                q = ll % 2

                if ll < 7:
                    @pl.when(ll + 1 < nvalid)
                    def _(ll=ll, q=q):
                        pltpu.async_copy(scr.at[idxf[ll + 1]], rows[1 - q],
                                         sg[1 - q])

                @pl.when(ll < nvalid)
                def _(ll=ll, q=q):
                    pltpu.make_async_copy(
                        scr.at[idxf[ll]], rows[q], sg[q]).wait()
                    b_compute(q, ll)

                    @pl.when((j > 0) | (ll >= 2))
                    def _(ll=ll, q=q):
                        out_wait(q, k, ll)

                    out_issue(q, k, ll)
        return carry

    lax.fori_loop(0, SPT // 2, b_major, 0)
    k_last = kbase + SPT - 1
    out_wait(0, k_last, 0)
    out_wait(1, k_last, 1)


@jax.jit
def kernel(input, emb_weight):
    idx_t = jnp.pad(input.T, ((0, LPAD - L), (0, 0)))
    tbl_t = emb_weight.T
    tail_t = jnp.pad(emb_weight.T[:, NFULL * WA:],
                     ((0, 0), (0, WA - ATAIL)))
    mesh = plsc.VectorSubcoreMesh(core_axis_name="c", subcore_axis_name="s")
    out = pl.kernel(
        _body,
        mesh=mesh,
        compiler_params=pltpu.CompilerParams(
            use_tc_tiling_on_sc=True, needs_layout_passes=False),
        out_type=jax.ShapeDtypeStruct((L, DIM, B), jnp.float32),
        scratch_types=[
            pltpu.HBM((250000, 128), jnp.float32),
            pltpu.VMEM((HALF, WA), jnp.float32),
            pltpu.VMEM((HALF, WA), jnp.float32),
            pltpu.VMEM((HALF, WA), jnp.float32),
            pltpu.VMEM((HALF, WA), jnp.float32),
            pltpu.VMEM((8, CB), jnp.int32),
            pltpu.VMEM((8, CB), jnp.int32),
            *[pltpu.VMEM((CB,), jnp.int32) for _ in range(16)],
            pltpu.VMEM((CB, 128), jnp.float32),
            pltpu.VMEM((CB, 128), jnp.float32),
            pltpu.VMEM((HALF, CB), jnp.float32),
            pltpu.VMEM((HALF, CB), jnp.float32),
            pltpu.VMEM((DIM, WA), jnp.float32),
            pltpu.VMEM((8, WA), jnp.float32),
            *[pltpu.SemaphoreType.DMA for _ in range(10)],
        ],
    )(idx_t, tbl_t, tail_t)
    return out.transpose(2, 0, 1)
